# baseline (speedup 1.0000x reference)
import jax
import jax.numpy as jnp
from jax import lax
from jax.experimental import pallas as pl
from jax.experimental.pallas import tpu as pltpu
from jax.experimental.pallas import tpu_sc as plsc

N_NODES = 10000
N_EDGES = 320000
D_FEAT = 128

NC = 2
NS = 16
L = 16

EPT = N_EDGES // (NC * NS)
K = 80
NCHUNK = EPT // K
_CPR = 1000
RPT = N_NODES // NS


def _sc_body(x, esrc, edst, part, sbuf, dbuf, rows0, rows1, acc_sh,
             semi, sem0, sem1, ssem0, ssem1):
    c = lax.axis_index("c")
    s = lax.axis_index("s")
    wid = c * NS + s

    da = pltpu.async_copy(esrc.at[wid], sbuf, semi)
    db = pltpu.async_copy(edst.at[wid], dbuf, semi)

    zv = jnp.zeros((L,), jnp.float32)

    @pl.loop(0, K)
    def _zero(r):
        for j in range(D_FEAT // L):
            rows0[r, pl.ds(j * L, L)] = zv

    row0 = s * RPT

    @pl.loop(0, RPT // K)
    def _zcopy(i):
        pltpu.sync_copy(rows0, acc_sh.at[pl.ds(row0 + i * K, K)])

    pltpu.sync_copy(rows0.at[pl.ds(0, RPT - (RPT // K) * K)],
                    acc_sh.at[pl.ds(row0 + (RPT // K) * K,
                                    RPT - (RPT // K) * K)])

    da.wait()
    db.wait()
    plsc.subcore_barrier()

    pltpu.async_copy(x.at[sbuf.at[pl.ds(0, K)]], rows0, sem0)
    pltpu.async_copy(x.at[sbuf.at[pl.ds(K, K)]], rows1, sem1)

    @pl.loop(0, NCHUNK - 1, step=2)
    def _edges(i):
        pltpu.make_async_copy(x.at[pl.ds(0, K)], rows0, sem0).wait()
        d0 = pltpu.async_copy(rows0, acc_sh.at[dbuf.at[i]], ssem0, add=True)
        pltpu.make_async_copy(x.at[pl.ds(0, K)], rows1, sem1).wait()
        d1 = pltpu.async_copy(rows1, acc_sh.at[dbuf.at[i + 1]], ssem1,
                              add=True)
        d0.wait()
        pltpu.async_copy(x.at[sbuf.at[pl.ds((i + 2) * K, K)]], rows0, sem0)
        d1.wait()

        @pl.when(i + 3 < NCHUNK)
        def _prefetch():
            pltpu.async_copy(x.at[sbuf.at[pl.ds((i + 3) * K, K)]],
                             rows1, sem1)

    pltpu.make_async_copy(x.at[pl.ds(0, K)], rows0, sem0).wait()
    pltpu.sync_copy(rows0, acc_sh.at[dbuf.at[NCHUNK - 1]], add=True)

    plsc.subcore_barrier()

    @pl.when(s < N_NODES // _CPR)
    def _writeout():
        r0 = s * _CPR
        pltpu.sync_copy(acc_sh.at[pl.ds(r0, _CPR)],
                        part.at[c, pl.ds(r0, _CPR)])


def _combine_body(dt_ref, x_ref, p_ref, o_ref):
    o_ref[...] = x_ref[...] + dt_ref[0] * (p_ref[0] + p_ref[1])


_BLK = 1000


def kernel(x, edge_index, start, end):
    esrc = edge_index[0].reshape(NC * NS, EPT)
    edst = edge_index[1].reshape(NC * NS, NCHUNK, K)

    part = pl.kernel(
        _sc_body,
        out_type=jax.ShapeDtypeStruct((NC, N_NODES, D_FEAT), jnp.float32),
        mesh=plsc.VectorSubcoreMesh(
            core_axis_name="c", subcore_axis_name="s",
            num_cores=NC, num_subcores=NS),
        scratch_types=[
            pltpu.VMEM((EPT,), jnp.int32),
            pltpu.VMEM((NCHUNK, K), jnp.int32),
            pltpu.VMEM((K, D_FEAT), jnp.float32),
            pltpu.VMEM((K, D_FEAT), jnp.float32),
            pltpu.VMEM_SHARED((N_NODES, D_FEAT), jnp.float32),
            pltpu.SemaphoreType.DMA,
            pltpu.SemaphoreType.DMA,
            pltpu.SemaphoreType.DMA,
            pltpu.SemaphoreType.DMA,
            pltpu.SemaphoreType.DMA,
        ],
    )(x, esrc, edst)

    dt = jnp.reshape(end - start, (1,)).astype(jnp.float32)
    out = pl.pallas_call(
        _combine_body,
        out_shape=jax.ShapeDtypeStruct((N_NODES, D_FEAT), jnp.float32),
        grid=(N_NODES // _BLK,),
        in_specs=[
            pl.BlockSpec(memory_space=pltpu.SMEM),
            pl.BlockSpec((_BLK, D_FEAT), lambda i: (i, 0)),
            pl.BlockSpec((NC, _BLK, D_FEAT), lambda i: (0, i, 0)),
        ],
        out_specs=pl.BlockSpec((_BLK, D_FEAT), lambda i: (i, 0)),
    )(dt, x, part)
    return out


# speedup vs baseline: 1.2305x; 1.2305x over previous
import jax
import jax.numpy as jnp
from jax import lax
from jax.experimental import pallas as pl
from jax.experimental.pallas import tpu as pltpu
from jax.experimental.pallas import tpu_sc as plsc

N_NODES = 10000
N_EDGES = 320000
D_FEAT = 128

NC = 2
NS = 16
L = 16

EPT = N_EDGES // (NC * NS)
K = 80
NCHUNK = EPT // K
_CPR = 1000
RPT = N_NODES // NS


def _sc_body(x, esrc, edst, part, sbuf, dbuf, rows0, rows1, acc_sh,
             semi, sem0, sem1):
    c = lax.axis_index("c")
    s = lax.axis_index("s")
    wid = c * NS + s

    da = pltpu.async_copy(esrc.at[wid], sbuf, semi)
    db = pltpu.async_copy(edst.at[wid], dbuf, semi)

    zv = jnp.zeros((L,), jnp.float32)

    @pl.loop(0, K)
    def _zero(r):
        for j in range(D_FEAT // L):
            rows0[r, pl.ds(j * L, L)] = zv

    row0 = s * RPT

    @pl.loop(0, RPT // K)
    def _zcopy(i):
        pltpu.sync_copy(rows0, acc_sh.at[pl.ds(row0 + i * K, K)])

    pltpu.sync_copy(rows0.at[pl.ds(0, RPT - (RPT // K) * K)],
                    acc_sh.at[pl.ds(row0 + (RPT // K) * K,
                                    RPT - (RPT // K) * K)])

    da.wait()
    db.wait()
    plsc.subcore_barrier()

    pltpu.async_copy(x.at[sbuf.at[pl.ds(0, K)]], rows0, sem0)

    @pl.loop(0, NCHUNK - 1, step=2)
    def _edges(i):
        d1 = pltpu.async_copy(x.at[sbuf.at[pl.ds((i + 1) * K, K)]],
                              rows1, sem1)
        pltpu.make_async_copy(x.at[pl.ds(0, K)], rows0, sem0).wait()
        pltpu.sync_copy(rows0, acc_sh.at[dbuf.at[i]], add=True)
        pltpu.async_copy(x.at[sbuf.at[pl.ds((i + 2) * K, K)]], rows0, sem0)
        d1.wait()
        pltpu.sync_copy(rows1, acc_sh.at[dbuf.at[i + 1]], add=True)

    pltpu.make_async_copy(x.at[pl.ds(0, K)], rows0, sem0).wait()
    pltpu.sync_copy(rows0, acc_sh.at[dbuf.at[NCHUNK - 1]], add=True)

    plsc.subcore_barrier()

    @pl.when(s < N_NODES // _CPR)
    def _writeout():
        r0 = s * _CPR
        pltpu.sync_copy(acc_sh.at[pl.ds(r0, _CPR)],
                        part.at[c, pl.ds(r0, _CPR)])


def _combine_body(dt_ref, x_ref, p_ref, o_ref):
    o_ref[...] = x_ref[...] + dt_ref[0] * (p_ref[0] + p_ref[1])


_BLK = 1000


def kernel(x, edge_index, start, end):
    esrc = edge_index[0].reshape(NC * NS, EPT)
    edst = edge_index[1].reshape(NC * NS, NCHUNK, K)

    part = pl.kernel(
        _sc_body,
        out_type=jax.ShapeDtypeStruct((NC, N_NODES, D_FEAT), jnp.float32),
        mesh=plsc.VectorSubcoreMesh(
            core_axis_name="c", subcore_axis_name="s",
            num_cores=NC, num_subcores=NS),
        scratch_types=[
            pltpu.VMEM((EPT,), jnp.int32),
            pltpu.VMEM((NCHUNK, K), jnp.int32),
            pltpu.VMEM((K, D_FEAT), jnp.float32),
            pltpu.VMEM((K, D_FEAT), jnp.float32),
            pltpu.VMEM_SHARED((N_NODES, D_FEAT), jnp.float32),
            pltpu.SemaphoreType.DMA,
            pltpu.SemaphoreType.DMA,
            pltpu.SemaphoreType.DMA,
        ],
    )(x, esrc, edst)

    dt = jnp.reshape(end - start, (1,)).astype(jnp.float32)
    out = pl.pallas_call(
        _combine_body,
        out_shape=jax.ShapeDtypeStruct((N_NODES, D_FEAT), jnp.float32),
        grid=(N_NODES // _BLK,),
        in_specs=[
            pl.BlockSpec(memory_space=pltpu.SMEM),
            pl.BlockSpec((_BLK, D_FEAT), lambda i: (i, 0)),
            pl.BlockSpec((NC, _BLK, D_FEAT), lambda i: (0, i, 0)),
        ],
        out_specs=pl.BlockSpec((_BLK, D_FEAT), lambda i: (i, 0)),
    )(dt, x, part)
    return out


# speedup vs baseline: 1.2765x; 1.0374x over previous
import jax
import jax.numpy as jnp
from jax import lax
from jax.experimental import pallas as pl
from jax.experimental.pallas import tpu as pltpu
from jax.experimental.pallas import tpu_sc as plsc

N_NODES = 10000
N_EDGES = 320000
D_FEAT = 128

NC = 2
NS = 16
L = 16

EPT = N_EDGES // (NC * NS)
K = 80
NCHUNK = EPT // K
_CPR = 1000
RPT = N_NODES // NS


def _sc_body(x, esrc, edst, part, sbuf, dbuf, rows0, rows1, acc_sh,
             semi, sem0, sem1):
    c = lax.axis_index("c")
    s = lax.axis_index("s")
    wid = c * NS + s

    da = pltpu.async_copy(esrc.at[0, wid], sbuf, semi)
    db = pltpu.async_copy(edst.at[1, wid], dbuf, semi)

    zv = jnp.zeros((L,), jnp.float32)

    @pl.loop(0, K)
    def _zero(r):
        for j in range(D_FEAT // L):
            rows0[r, pl.ds(j * L, L)] = zv

    row0 = s * RPT

    @pl.loop(0, RPT // K)
    def _zcopy(i):
        pltpu.sync_copy(rows0, acc_sh.at[pl.ds(row0 + i * K, K)])

    pltpu.sync_copy(rows0.at[pl.ds(0, RPT - (RPT // K) * K)],
                    acc_sh.at[pl.ds(row0 + (RPT // K) * K,
                                    RPT - (RPT // K) * K)])

    da.wait()
    db.wait()
    plsc.subcore_barrier()

    pltpu.async_copy(x.at[sbuf.at[pl.ds(0, K)]], rows0, sem0)

    @pl.loop(0, NCHUNK - 1, step=2)
    def _edges(i):
        d1 = pltpu.async_copy(x.at[sbuf.at[pl.ds((i + 1) * K, K)]],
                              rows1, sem1)
        pltpu.make_async_copy(x.at[pl.ds(0, K)], rows0, sem0).wait()
        pltpu.sync_copy(rows0, acc_sh.at[dbuf.at[i]], add=True)
        pltpu.async_copy(x.at[sbuf.at[pl.ds((i + 2) * K, K)]], rows0, sem0)
        d1.wait()
        pltpu.sync_copy(rows1, acc_sh.at[dbuf.at[i + 1]], add=True)

    pltpu.make_async_copy(x.at[pl.ds(0, K)], rows0, sem0).wait()
    pltpu.sync_copy(rows0, acc_sh.at[dbuf.at[NCHUNK - 1]], add=True)

    plsc.subcore_barrier()

    @pl.when(s < N_NODES // _CPR)
    def _writeout():
        r0 = s * _CPR
        pltpu.sync_copy(acc_sh.at[pl.ds(r0, _CPR)],
                        part.at[c, pl.ds(r0, _CPR)])


def _combine_body(se_ref, x_ref, p_ref, o_ref):
    dt = se_ref[1] - se_ref[0]
    o_ref[...] = x_ref[...] + dt * (p_ref[0] + p_ref[1])


_BLK = 1000


def kernel(x, edge_index, start, end):
    esrc = edge_index.reshape(2, NC * NS, EPT)
    edst = edge_index.reshape(2, NC * NS, NCHUNK, K)

    part = pl.kernel(
        _sc_body,
        out_type=jax.ShapeDtypeStruct((NC, N_NODES, D_FEAT), jnp.float32),
        mesh=plsc.VectorSubcoreMesh(
            core_axis_name="c", subcore_axis_name="s",
            num_cores=NC, num_subcores=NS),
        scratch_types=[
            pltpu.VMEM((EPT,), jnp.int32),
            pltpu.VMEM((NCHUNK, K), jnp.int32),
            pltpu.VMEM((K, D_FEAT), jnp.float32),
            pltpu.VMEM((K, D_FEAT), jnp.float32),
            pltpu.VMEM_SHARED((N_NODES, D_FEAT), jnp.float32),
            pltpu.SemaphoreType.DMA,
            pltpu.SemaphoreType.DMA,
            pltpu.SemaphoreType.DMA,
        ],
    )(x, esrc, edst)

    se = jnp.stack([start, end]).astype(jnp.float32)
    out = pl.pallas_call(
        _combine_body,
        out_shape=jax.ShapeDtypeStruct((N_NODES, D_FEAT), jnp.float32),
        grid=(N_NODES // _BLK,),
        in_specs=[
            pl.BlockSpec(memory_space=pltpu.SMEM),
            pl.BlockSpec((_BLK, D_FEAT), lambda i: (i, 0)),
            pl.BlockSpec((NC, _BLK, D_FEAT), lambda i: (0, i, 0)),
        ],
        out_specs=pl.BlockSpec((_BLK, D_FEAT), lambda i: (i, 0)),
    )(se, x, part)
    return out


# speedup vs baseline: 1.4404x; 1.1284x over previous
import jax
import jax.numpy as jnp
from jax import lax
from jax.experimental import pallas as pl
from jax.experimental.pallas import tpu as pltpu
from jax.experimental.pallas import tpu_sc as plsc

N_NODES = 10000
N_EDGES = 320000
D_FEAT = 128

NC = 2
NS = 16
L = 16

K = 128
NCH = N_EDGES // K
CPT = NCH // (NC * NS)
G = 6
NG = CPT // G
_CPR = 1000


def _copy_dst_to_cbuf(pbuf, cbuf, j):
    for k in range(K // L):
        cbuf[pl.ds(k * L, L)] = pbuf[1, pl.ds(j * K + k * L, L)]


def _sc_body(x, eidx, part, pbufa, pbufb, rows0, rows1, cbuf, acc_sh,
             isema, isemb, sem0, sem1):
    c = lax.axis_index("c")
    s = lax.axis_index("s")
    wid = c * NS + s
    c0 = wid * CPT

    rows = (rows0, rows1)
    sems = (sem0, sem1)
    pbufs = (pbufa, pbufb)
    isems = (isema, isemb)

    def idx_prefetch(g, b):
        off = (c0 + g * G) * K
        pltpu.async_copy(eidx.at[:, pl.ds(off, G * K)], pbufs[b], isems[b])

    def issue_gather(pb, j, rb):
        pltpu.async_copy(x.at[pb.at[0, pl.ds(j * K, K)]], rows[rb], sems[rb])

    def wait_gather(rb):
        pltpu.make_async_copy(x.at[pl.ds(0, K)], rows[rb], sems[rb]).wait()

    def wait_idx(b):
        pltpu.make_async_copy(eidx.at[:, pl.ds(0, G * K)], pbufs[b],
                              isems[b]).wait()

    zv = jnp.zeros((L,), jnp.float32)

    @pl.loop(0, K)
    def _zero(r):
        for j in range(D_FEAT // L):
            rows0[r, pl.ds(j * L, L)] = zv

    row0 = s * (N_NODES // NS)
    nfull = (N_NODES // NS) // K

    @pl.loop(0, nfull)
    def _zcopy(i):
        pltpu.sync_copy(rows0, acc_sh.at[pl.ds(row0 + i * K, K)])

    rem = N_NODES // NS - nfull * K
    pltpu.sync_copy(rows0.at[pl.ds(0, rem)],
                    acc_sh.at[pl.ds(row0 + nfull * K, rem)])

    idx_prefetch(0, 0)
    idx_prefetch(1, 1)
    plsc.subcore_barrier()

    wait_idx(0)
    issue_gather(pbufa, 0, 0)

    def process_group(p, b, g, has_next):
        pb = pbufs[b]
        qb = pbufs[1 - b]
        for j in range(G):
            if j + 1 < G:
                issue_gather(pb, j + 1, (j + 1) % 2)
            elif has_next:
                wait_idx(1 - b)
                issue_gather(qb, 0, 0)
            _copy_dst_to_cbuf(pb, cbuf, j)
            wait_gather(j % 2)
            if j == G - 1 and has_next:
                @pl.when(g + 2 < NG)
                def _pf():
                    idx_prefetch(g + 2, b)
            pltpu.sync_copy(rows[j % 2], acc_sh.at[cbuf], add=True)

    @pl.loop(0, NG // 2)
    def _pairs(p):
        process_group(p, 0, 2 * p, True)
        process_group(p, 1, 2 * p + 1, True)

    process_group(0, 0, NG - 1, False)

    @pl.when(wid < NCH - NC * NS * CPT)
    def _extra():
        off = (NC * NS * CPT + wid) * K
        pltpu.sync_copy(eidx.at[:, pl.ds(off, K)],
                        pbufa.at[:, pl.ds(0, K)])
        pltpu.async_copy(x.at[pbufa.at[0, pl.ds(0, K)]], rows0, sem0)
        _copy_dst_to_cbuf(pbufa, cbuf, 0)
        wait_gather(0)
        pltpu.sync_copy(rows0, acc_sh.at[cbuf], add=True)

    plsc.subcore_barrier()

    @pl.when(s < N_NODES // _CPR)
    def _writeout():
        r0 = s * _CPR
        pltpu.sync_copy(acc_sh.at[pl.ds(r0, _CPR)],
                        part.at[c, pl.ds(r0, _CPR)])


def _combine_body(se_ref, x_ref, p_ref, o_ref):
    dt = se_ref[1] - se_ref[0]
    o_ref[...] = x_ref[...] + dt * (p_ref[0] + p_ref[1])


_BLK = 1000


def kernel(x, edge_index, start, end):
    part = pl.kernel(
        _sc_body,
        out_type=jax.ShapeDtypeStruct((NC, N_NODES, D_FEAT), jnp.float32),
        mesh=plsc.VectorSubcoreMesh(
            core_axis_name="c", subcore_axis_name="s",
            num_cores=NC, num_subcores=NS),
        scratch_types=[
            pltpu.VMEM((2, G * K), jnp.int32),
            pltpu.VMEM((2, G * K), jnp.int32),
            pltpu.VMEM((K, D_FEAT), jnp.float32),
            pltpu.VMEM((K, D_FEAT), jnp.float32),
            pltpu.VMEM((K,), jnp.int32),
            pltpu.VMEM_SHARED((N_NODES, D_FEAT), jnp.float32),
            pltpu.SemaphoreType.DMA,
            pltpu.SemaphoreType.DMA,
            pltpu.SemaphoreType.DMA,
            pltpu.SemaphoreType.DMA,
        ],
    )(x, edge_index)

    se = jnp.stack([start, end]).astype(jnp.float32)
    out = pl.pallas_call(
        _combine_body,
        out_shape=jax.ShapeDtypeStruct((N_NODES, D_FEAT), jnp.float32),
        grid=(N_NODES // _BLK,),
        in_specs=[
            pl.BlockSpec(memory_space=pltpu.SMEM),
            pl.BlockSpec((_BLK, D_FEAT), lambda i: (i, 0)),
            pl.BlockSpec((NC, _BLK, D_FEAT), lambda i: (0, i, 0)),
        ],
        out_specs=pl.BlockSpec((_BLK, D_FEAT), lambda i: (i, 0)),
    )(se, x, part)
    return out


# speedup vs baseline: 1.4781x; 1.0262x over previous
import jax
import jax.numpy as jnp
from jax import lax
from jax.experimental import pallas as pl
from jax.experimental.pallas import tpu as pltpu
from jax.experimental.pallas import tpu_sc as plsc

N_NODES = 10000
N_EDGES = 320000
D_FEAT = 128

NC = 2
NS = 16
L = 16

K = 128
NCH = N_EDGES // K
CPT = NCH // (NC * NS)
G = 6
NG = CPT // G
_CPR = 1000


def _copy_dst_to_cbuf(pbuf, cbuf, j):
    for k in range(K // L):
        cbuf[pl.ds(k * L, L)] = pbuf[1, pl.ds(j * K + k * L, L)]


def _sc_body(x, eidx, part, pbufa, pbufb, rows0, rows1, cbuf, acc_sh,
             isema, isemb, sem0, sem1):
    c = lax.axis_index("c")
    s = lax.axis_index("s")
    wid = c * NS + s
    c0 = wid * CPT

    rows = (rows0, rows1)
    sems = (sem0, sem1)
    pbufs = (pbufa, pbufb)
    isems = (isema, isemb)

    def idx_prefetch(g, b):
        off = (c0 + g * G) * K
        pltpu.async_copy(eidx.at[:, pl.ds(off, G * K)], pbufs[b], isems[b])

    def issue_gather(pb, j, rb):
        pltpu.async_copy(x.at[pb.at[0, pl.ds(j * K, K)]], rows[rb], sems[rb])

    def wait_gather(rb):
        pltpu.make_async_copy(x.at[pl.ds(0, K)], rows[rb], sems[rb]).wait()

    def wait_idx(b):
        pltpu.make_async_copy(eidx.at[:, pl.ds(0, G * K)], pbufs[b],
                              isems[b]).wait()

    idx_prefetch(0, 0)
    idx_prefetch(1, 1)

    zv = jnp.zeros((L,), jnp.float32)

    @pl.loop(0, K)
    def _zero(r):
        for j in range(D_FEAT // L):
            rows0[r, pl.ds(j * L, L)] = zv

    row0 = s * (N_NODES // NS)
    nfull = (N_NODES // NS) // K

    @pl.loop(0, nfull)
    def _zcopy(i):
        pltpu.sync_copy(rows0, acc_sh.at[pl.ds(row0 + i * K, K)])

    rem = N_NODES // NS - nfull * K
    pltpu.sync_copy(rows0.at[pl.ds(0, rem)],
                    acc_sh.at[pl.ds(row0 + nfull * K, rem)])

    plsc.subcore_barrier()

    wait_idx(0)
    issue_gather(pbufa, 0, 0)

    def process_group(p, b, g, has_next):
        pb = pbufs[b]
        qb = pbufs[1 - b]
        for j in range(G):
            if j + 1 < G:
                issue_gather(pb, j + 1, (j + 1) % 2)
            elif has_next:
                wait_idx(1 - b)
                issue_gather(qb, 0, 0)
            _copy_dst_to_cbuf(pb, cbuf, j)
            wait_gather(j % 2)
            if j == G - 1 and has_next:
                @pl.when(g + 2 < NG)
                def _pf():
                    idx_prefetch(g + 2, b)
            pltpu.sync_copy(rows[j % 2], acc_sh.at[cbuf], add=True)

    @pl.loop(0, NG // 2)
    def _pairs(p):
        process_group(p, 0, 2 * p, True)
        process_group(p, 1, 2 * p + 1, True)

    process_group(0, 0, NG - 1, False)

    @pl.when(wid < NCH - NC * NS * CPT)
    def _extra():
        off = (NC * NS * CPT + wid) * K
        pltpu.sync_copy(eidx.at[:, pl.ds(off, K)],
                        pbufa.at[:, pl.ds(0, K)])
        pltpu.async_copy(x.at[pbufa.at[0, pl.ds(0, K)]], rows0, sem0)
        _copy_dst_to_cbuf(pbufa, cbuf, 0)
        wait_gather(0)
        pltpu.sync_copy(rows0, acc_sh.at[cbuf], add=True)

    plsc.subcore_barrier()

    @pl.when(s < N_NODES // _CPR)
    def _writeout():
        r0 = s * _CPR
        pltpu.sync_copy(acc_sh.at[pl.ds(r0, _CPR)],
                        part.at[c, pl.ds(r0, _CPR)])


def _combine_body(se_ref, x_ref, p_ref, o_ref):
    dt = se_ref[1] - se_ref[0]
    o_ref[...] = x_ref[...] + dt * (p_ref[0] + p_ref[1])


_BLK = 2000


def kernel(x, edge_index, start, end):
    part = pl.kernel(
        _sc_body,
        out_type=jax.ShapeDtypeStruct((NC, N_NODES, D_FEAT), jnp.float32),
        mesh=plsc.VectorSubcoreMesh(
            core_axis_name="c", subcore_axis_name="s",
            num_cores=NC, num_subcores=NS),
        scratch_types=[
            pltpu.VMEM((2, G * K), jnp.int32),
            pltpu.VMEM((2, G * K), jnp.int32),
            pltpu.VMEM((K, D_FEAT), jnp.float32),
            pltpu.VMEM((K, D_FEAT), jnp.float32),
            pltpu.VMEM((K,), jnp.int32),
            pltpu.VMEM_SHARED((N_NODES, D_FEAT), jnp.float32),
            pltpu.SemaphoreType.DMA,
            pltpu.SemaphoreType.DMA,
            pltpu.SemaphoreType.DMA,
            pltpu.SemaphoreType.DMA,
        ],
    )(x, edge_index)

    se = jnp.stack([start, end]).astype(jnp.float32)
    out = pl.pallas_call(
        _combine_body,
        out_shape=jax.ShapeDtypeStruct((N_NODES, D_FEAT), jnp.float32),
        grid=(N_NODES // _BLK,),
        in_specs=[
            pl.BlockSpec(memory_space=pltpu.SMEM),
            pl.BlockSpec((_BLK, D_FEAT), lambda i: (i, 0)),
            pl.BlockSpec((NC, _BLK, D_FEAT), lambda i: (0, i, 0)),
        ],
        out_specs=pl.BlockSpec((_BLK, D_FEAT), lambda i: (i, 0)),
    )(se, x, part)
    return out


# speedup vs baseline: 1.4868x; 1.0059x over previous
import jax
import jax.numpy as jnp
from jax import lax
from jax.experimental import pallas as pl
from jax.experimental.pallas import tpu as pltpu
from jax.experimental.pallas import tpu_sc as plsc

N_NODES = 10000
N_EDGES = 320000
D_FEAT = 128

NC = 2
NS = 16
L = 16

K = 128
NCH = N_EDGES // K
CPT = NCH // (NC * NS)
G = 6
NG = CPT // G
_CPR = 1000


def _copy_dst_to_cbuf(pbuf, cbuf, j):
    for k in range(K // L):
        cbuf[pl.ds(k * L, L)] = pbuf[1, pl.ds(j * K + k * L, L)]


def _sc_body(x, eidx, part, pbufa, pbufb, rows0, rows1, cbuf, acc_sh,
             isema, isemb, sem0, sem1):
    c = lax.axis_index("c")
    s = lax.axis_index("s")
    wid = c * NS + s
    c0 = wid * CPT

    rows = (rows0, rows1)
    sems = (sem0, sem1)
    pbufs = (pbufa, pbufb)
    isems = (isema, isemb)

    def idx_prefetch(g, b):
        off = (c0 + g * G) * K
        pltpu.async_copy(eidx.at[:, pl.ds(off, G * K)], pbufs[b], isems[b])

    def issue_gather(pb, j, rb):
        pltpu.async_copy(x.at[pb.at[0, pl.ds(j * K, K)]], rows[rb], sems[rb])

    def wait_gather(rb):
        pltpu.make_async_copy(x.at[pl.ds(0, K)], rows[rb], sems[rb]).wait()

    def wait_idx(b):
        pltpu.make_async_copy(eidx.at[:, pl.ds(0, G * K)], pbufs[b],
                              isems[b]).wait()

    idx_prefetch(0, 0)
    idx_prefetch(1, 1)

    zv = jnp.zeros((L,), jnp.float32)

    @pl.loop(0, K)
    def _zero(r):
        for j in range(D_FEAT // L):
            rows0[r, pl.ds(j * L, L)] = zv

    row0 = s * (N_NODES // NS)
    nfull = (N_NODES // NS) // K

    @pl.loop(0, nfull)
    def _zcopy(i):
        pltpu.sync_copy(rows0, acc_sh.at[pl.ds(row0 + i * K, K)])

    rem = N_NODES // NS - nfull * K
    pltpu.sync_copy(rows0.at[pl.ds(0, rem)],
                    acc_sh.at[pl.ds(row0 + nfull * K, rem)])

    wait_idx(0)
    issue_gather(pbufa, 0, 0)
    plsc.subcore_barrier()

    def process_group(p, b, g, has_next):
        pb = pbufs[b]
        qb = pbufs[1 - b]
        for j in range(G):
            if j + 1 < G:
                issue_gather(pb, j + 1, (j + 1) % 2)
            elif has_next:
                wait_idx(1 - b)
                issue_gather(qb, 0, 0)
            _copy_dst_to_cbuf(pb, cbuf, j)
            wait_gather(j % 2)
            if j == G - 1 and has_next:
                @pl.when(g + 2 < NG)
                def _pf():
                    idx_prefetch(g + 2, b)
            pltpu.sync_copy(rows[j % 2], acc_sh.at[cbuf], add=True)

    @pl.loop(0, NG // 2)
    def _pairs(p):
        process_group(p, 0, 2 * p, True)
        process_group(p, 1, 2 * p + 1, True)

    process_group(0, 0, NG - 1, False)

    @pl.when(wid < NCH - NC * NS * CPT)
    def _extra():
        off = (NC * NS * CPT + wid) * K
        pltpu.sync_copy(eidx.at[:, pl.ds(off, K)],
                        pbufa.at[:, pl.ds(0, K)])
        pltpu.async_copy(x.at[pbufa.at[0, pl.ds(0, K)]], rows0, sem0)
        _copy_dst_to_cbuf(pbufa, cbuf, 0)
        wait_gather(0)
        pltpu.sync_copy(rows0, acc_sh.at[cbuf], add=True)

    plsc.subcore_barrier()

    @pl.when(s < N_NODES // _CPR)
    def _writeout():
        r0 = s * _CPR
        pltpu.sync_copy(acc_sh.at[pl.ds(r0, _CPR)],
                        part.at[c, pl.ds(r0, _CPR)])


def _combine_body(se_ref, x_ref, p_ref, o_ref):
    dt = se_ref[1] - se_ref[0]
    o_ref[...] = x_ref[...] + dt * (p_ref[0] + p_ref[1])


_BLK = 5000


def kernel(x, edge_index, start, end):
    part = pl.kernel(
        _sc_body,
        out_type=jax.ShapeDtypeStruct((NC, N_NODES, D_FEAT), jnp.float32),
        mesh=plsc.VectorSubcoreMesh(
            core_axis_name="c", subcore_axis_name="s",
            num_cores=NC, num_subcores=NS),
        scratch_types=[
            pltpu.VMEM((2, G * K), jnp.int32),
            pltpu.VMEM((2, G * K), jnp.int32),
            pltpu.VMEM((K, D_FEAT), jnp.float32),
            pltpu.VMEM((K, D_FEAT), jnp.float32),
            pltpu.VMEM((K,), jnp.int32),
            pltpu.VMEM_SHARED((N_NODES, D_FEAT), jnp.float32),
            pltpu.SemaphoreType.DMA,
            pltpu.SemaphoreType.DMA,
            pltpu.SemaphoreType.DMA,
            pltpu.SemaphoreType.DMA,
        ],
    )(x, edge_index)

    se = jnp.stack([start, end]).astype(jnp.float32)
    out = pl.pallas_call(
        _combine_body,
        out_shape=jax.ShapeDtypeStruct((N_NODES, D_FEAT), jnp.float32),
        grid=(N_NODES // _BLK,),
        in_specs=[
            pl.BlockSpec(memory_space=pltpu.SMEM),
            pl.BlockSpec((_BLK, D_FEAT), lambda i: (i, 0)),
            pl.BlockSpec((NC, _BLK, D_FEAT), lambda i: (0, i, 0)),
        ],
        out_specs=pl.BlockSpec((_BLK, D_FEAT), lambda i: (i, 0)),
    )(se, x, part)
    return out


# speedup vs baseline: 1.5070x; 1.0136x over previous
import jax
import jax.numpy as jnp
from jax import lax
from jax.experimental import pallas as pl
from jax.experimental.pallas import tpu as pltpu
from jax.experimental.pallas import tpu_sc as plsc

N_NODES = 10000
N_EDGES = 320000
D_FEAT = 128

NC = 2
NS = 16
L = 16

K = 128
NCH = N_EDGES // K
CPT = NCH // (NC * NS)
G = 6
NG = CPT // G
_CPR = 1000


def _copy_dst_to_cbuf(pbuf, cbuf, j):
    for k in range(K // L):
        cbuf[pl.ds(k * L, L)] = pbuf[1, pl.ds(j * K + k * L, L)]


def _sc_body(x, eidx, part, pbufa, pbufb, rows0, rows1, cbuf, acc_sh,
             isema, isemb, sem0, sem1):
    c = lax.axis_index("c")
    s = lax.axis_index("s")
    wid = c * NS + s
    c0 = wid * CPT

    rows = (rows0, rows1)
    sems = (sem0, sem1)
    pbufs = (pbufa, pbufb)
    isems = (isema, isemb)

    def idx_prefetch(g, b):
        off = (c0 + g * G) * K
        pltpu.async_copy(eidx.at[:, pl.ds(off, G * K)], pbufs[b], isems[b])

    def issue_gather(pb, j, rb):
        pltpu.async_copy(x.at[pb.at[0, pl.ds(j * K, K)]], rows[rb], sems[rb])

    def wait_gather(rb):
        pltpu.make_async_copy(x.at[pl.ds(0, K)], rows[rb], sems[rb]).wait()

    def wait_idx(b):
        pltpu.make_async_copy(eidx.at[:, pl.ds(0, G * K)], pbufs[b],
                              isems[b]).wait()

    idx_prefetch(0, 0)
    idx_prefetch(1, 1)

    zv = jnp.zeros((L,), jnp.float32)

    @pl.loop(0, K)
    def _zero(r):
        for j in range(D_FEAT // L):
            rows1[r, pl.ds(j * L, L)] = zv

    wait_idx(0)
    issue_gather(pbufa, 0, 0)

    row0 = s * (N_NODES // NS)
    nfull = (N_NODES // NS) // K

    @pl.loop(0, nfull)
    def _zcopy(i):
        pltpu.sync_copy(rows1, acc_sh.at[pl.ds(row0 + i * K, K)])

    rem = N_NODES // NS - nfull * K
    pltpu.sync_copy(rows1.at[pl.ds(0, rem)],
                    acc_sh.at[pl.ds(row0 + nfull * K, rem)])

    plsc.subcore_barrier()

    def process_group(p, b, g, has_next):
        pb = pbufs[b]
        qb = pbufs[1 - b]
        for j in range(G):
            if j + 1 < G:
                issue_gather(pb, j + 1, (j + 1) % 2)
            elif has_next:
                wait_idx(1 - b)
                issue_gather(qb, 0, 0)
            _copy_dst_to_cbuf(pb, cbuf, j)
            wait_gather(j % 2)
            if j == G - 1 and has_next:
                @pl.when(g + 2 < NG)
                def _pf():
                    idx_prefetch(g + 2, b)
            pltpu.sync_copy(rows[j % 2], acc_sh.at[cbuf], add=True)

    @pl.loop(0, NG // 2)
    def _pairs(p):
        process_group(p, 0, 2 * p, True)
        process_group(p, 1, 2 * p + 1, True)

    process_group(0, 0, NG - 1, False)

    @pl.when(wid < NCH - NC * NS * CPT)
    def _extra():
        off = (NC * NS * CPT + wid) * K
        pltpu.sync_copy(eidx.at[:, pl.ds(off, K)],
                        pbufa.at[:, pl.ds(0, K)])
        pltpu.async_copy(x.at[pbufa.at[0, pl.ds(0, K)]], rows0, sem0)
        _copy_dst_to_cbuf(pbufa, cbuf, 0)
        wait_gather(0)
        pltpu.sync_copy(rows0, acc_sh.at[cbuf], add=True)

    plsc.subcore_barrier()

    @pl.when(s < N_NODES // _CPR)
    def _writeout():
        r0 = s * _CPR
        pltpu.sync_copy(acc_sh.at[pl.ds(r0, _CPR)],
                        part.at[c, pl.ds(r0, _CPR)])


def _combine_body(se_ref, x_ref, p_ref, o_ref):
    dt = se_ref[1] - se_ref[0]
    o_ref[...] = x_ref[...] + dt * (p_ref[0] + p_ref[1])


_BLK = 5000


def kernel(x, edge_index, start, end):
    part = pl.kernel(
        _sc_body,
        out_type=jax.ShapeDtypeStruct((NC, N_NODES, D_FEAT), jnp.float32),
        mesh=plsc.VectorSubcoreMesh(
            core_axis_name="c", subcore_axis_name="s",
            num_cores=NC, num_subcores=NS),
        scratch_types=[
            pltpu.VMEM((2, G * K), jnp.int32),
            pltpu.VMEM((2, G * K), jnp.int32),
            pltpu.VMEM((K, D_FEAT), jnp.float32),
            pltpu.VMEM((K, D_FEAT), jnp.float32),
            pltpu.VMEM((K,), jnp.int32),
            pltpu.VMEM_SHARED((N_NODES, D_FEAT), jnp.float32),
            pltpu.SemaphoreType.DMA,
            pltpu.SemaphoreType.DMA,
            pltpu.SemaphoreType.DMA,
            pltpu.SemaphoreType.DMA,
        ],
    )(x, edge_index)

    se = jnp.stack([start, end]).astype(jnp.float32)
    out = pl.pallas_call(
        _combine_body,
        out_shape=jax.ShapeDtypeStruct((N_NODES, D_FEAT), jnp.float32),
        grid=(N_NODES // _BLK,),
        in_specs=[
            pl.BlockSpec(memory_space=pltpu.SMEM),
            pl.BlockSpec((_BLK, D_FEAT), lambda i: (i, 0)),
            pl.BlockSpec((NC, _BLK, D_FEAT), lambda i: (0, i, 0)),
        ],
        out_specs=pl.BlockSpec((_BLK, D_FEAT), lambda i: (i, 0)),
    )(se, x, part)
    return out


# speedup vs baseline: 1.5096x; 1.0017x over previous
import jax
import jax.numpy as jnp
from jax import lax
from jax.experimental import pallas as pl
from jax.experimental.pallas import tpu as pltpu
from jax.experimental.pallas import tpu_sc as plsc

N_NODES = 10000
N_EDGES = 320000
D_FEAT = 128

NC = 2
NS = 16
L = 16

K = 128
NCH = N_EDGES // K
CPT = NCH // (NC * NS)
G = 6
NG = CPT // G
_CPR = 1000


def _copy_dst_to_cbuf(pbuf, cbuf, j):
    for k in range(K // L):
        cbuf[pl.ds(k * L, L)] = pbuf[1, pl.ds(j * K + k * L, L)]


def _sc_body(x, eidx, part, pbufa, pbufb, rows0, rows1, cbuf, ebuf, acc_sh,
             isema, isemb, sem0, sem1, esem):
    c = lax.axis_index("c")
    s = lax.axis_index("s")
    wid = c * NS + s
    c0 = wid * CPT

    rows = (rows0, rows1)
    sems = (sem0, sem1)
    pbufs = (pbufa, pbufb)
    isems = (isema, isemb)

    def idx_prefetch(g, b):
        off = (c0 + g * G) * K
        pltpu.async_copy(eidx.at[:, pl.ds(off, G * K)], pbufs[b], isems[b])

    def issue_gather(pb, j, rb):
        pltpu.async_copy(x.at[pb.at[0, pl.ds(j * K, K)]], rows[rb], sems[rb])

    def wait_gather(rb):
        pltpu.make_async_copy(x.at[pl.ds(0, K)], rows[rb], sems[rb]).wait()

    def wait_idx(b):
        pltpu.make_async_copy(eidx.at[:, pl.ds(0, G * K)], pbufs[b],
                              isems[b]).wait()

    idx_prefetch(0, 0)
    idx_prefetch(1, 1)
    nextra = NCH - NC * NS * CPT

    @pl.when(wid < nextra)
    def _extra_prefetch():
        off = (NC * NS * CPT + wid) * K
        pltpu.async_copy(eidx.at[:, pl.ds(off, K)], ebuf, esem)

    zv = jnp.zeros((L,), jnp.float32)

    @pl.loop(0, K)
    def _zero(r):
        for j in range(D_FEAT // L):
            rows1[r, pl.ds(j * L, L)] = zv

    wait_idx(0)
    issue_gather(pbufa, 0, 0)

    row0 = s * (N_NODES // NS)
    nfull = (N_NODES // NS) // K

    @pl.loop(0, nfull)
    def _zcopy(i):
        pltpu.sync_copy(rows1, acc_sh.at[pl.ds(row0 + i * K, K)])

    rem = N_NODES // NS - nfull * K
    pltpu.sync_copy(rows1.at[pl.ds(0, rem)],
                    acc_sh.at[pl.ds(row0 + nfull * K, rem)])

    plsc.subcore_barrier()

    def process_group(b, g, has_next, last=False):
        pb = pbufs[b]
        qb = pbufs[1 - b]
        for j in range(G):
            if j + 1 < G:
                issue_gather(pb, j + 1, (j + 1) % 2)
            elif has_next:
                wait_idx(1 - b)
                issue_gather(qb, 0, 0)
            elif last:
                @pl.when(wid < nextra)
                def _eg():
                    pltpu.make_async_copy(eidx.at[:, pl.ds(0, K)], ebuf,
                                          esem).wait()
                    issue_gather(ebuf, 0, 0)
            _copy_dst_to_cbuf(pb, cbuf, j)
            wait_gather(j % 2)
            if j == G - 1 and has_next:
                @pl.when(g + 2 < NG)
                def _pf():
                    idx_prefetch(g + 2, b)
            pltpu.sync_copy(rows[j % 2], acc_sh.at[cbuf], add=True)

    @pl.loop(0, NG // 2)
    def _pairs(p):
        process_group(0, 2 * p, True)
        process_group(1, 2 * p + 1, True)

    process_group(0, NG - 1, False, last=True)

    @pl.when(wid < nextra)
    def _extra():
        _copy_dst_to_cbuf(ebuf, cbuf, 0)
        wait_gather(0)
        pltpu.sync_copy(rows0, acc_sh.at[cbuf], add=True)

    plsc.subcore_barrier()

    @pl.when(s < N_NODES // _CPR)
    def _writeout():
        r0 = s * _CPR
        pltpu.sync_copy(acc_sh.at[pl.ds(r0, _CPR)],
                        part.at[c, pl.ds(r0, _CPR)])


def _combine_body(se_ref, x_ref, p_ref, o_ref):
    dt = se_ref[1] - se_ref[0]
    o_ref[...] = x_ref[...] + dt * (p_ref[0] + p_ref[1])


_BLK = 5000


def kernel(x, edge_index, start, end):
    part = pl.kernel(
        _sc_body,
        out_type=jax.ShapeDtypeStruct((NC, N_NODES, D_FEAT), jnp.float32),
        mesh=plsc.VectorSubcoreMesh(
            core_axis_name="c", subcore_axis_name="s",
            num_cores=NC, num_subcores=NS),
        scratch_types=[
            pltpu.VMEM((2, G * K), jnp.int32),
            pltpu.VMEM((2, G * K), jnp.int32),
            pltpu.VMEM((K, D_FEAT), jnp.float32),
            pltpu.VMEM((K, D_FEAT), jnp.float32),
            pltpu.VMEM((K,), jnp.int32),
            pltpu.VMEM((2, K), jnp.int32),
            pltpu.VMEM_SHARED((N_NODES, D_FEAT), jnp.float32),
            pltpu.SemaphoreType.DMA,
            pltpu.SemaphoreType.DMA,
            pltpu.SemaphoreType.DMA,
            pltpu.SemaphoreType.DMA,
            pltpu.SemaphoreType.DMA,
        ],
    )(x, edge_index)

    se = jnp.stack([start, end]).astype(jnp.float32)
    out = pl.pallas_call(
        _combine_body,
        out_shape=jax.ShapeDtypeStruct((N_NODES, D_FEAT), jnp.float32),
        grid=(N_NODES // _BLK,),
        in_specs=[
            pl.BlockSpec(memory_space=pltpu.SMEM),
            pl.BlockSpec((_BLK, D_FEAT), lambda i: (i, 0)),
            pl.BlockSpec((NC, _BLK, D_FEAT), lambda i: (0, i, 0)),
        ],
        out_specs=pl.BlockSpec((_BLK, D_FEAT), lambda i: (i, 0)),
    )(se, x, part)
    return out
